# baseline (speedup 1.0000x reference)
import functools

import jax
import jax.numpy as jnp
from jax import lax
from jax.experimental import pallas as pl
from jax.experimental.pallas import tpu as pltpu
from jax.experimental.pallas import tpu_sc as plsc

B = 800000
D = 64
TR = 8
TCOLS = 6250
CC = 2
OUT = CC * 128
NW = 32

_TC_BLK = 16384
_TC_GRID = 32
SPLIT = _TC_BLK * _TC_GRID
SC_CHUNK0 = SPLIT // (CC * 128)
NCHUNKS = B // (CC * 128)
SC_NCH = NCHUNKS - SC_CHUNK0
SLOTS = -(-SC_NCH // NW)
if SLOTS % 2:
    SLOTS += 1

_mesh = plsc.VectorSubcoreMesh(core_axis_name="c", subcore_axis_name="s")


@functools.partial(
    pl.kernel,
    mesh=_mesh,
    out_type=jax.ShapeDtypeStruct((B,), jnp.float32),
    scratch_types=[
        pltpu.VMEM((2, TR, CC, 8, 128), jnp.float32),
        pltpu.VMEM((2, TR, CC, 8, 128), jnp.float32),
        pltpu.VMEM((2, OUT), jnp.float32),
        pltpu.SemaphoreType.DMA,
        pltpu.SemaphoreType.DMA,
        pltpu.SemaphoreType.DMA,
        pltpu.SemaphoreType.DMA,
    ],
)
def _sc_part(va_hbm, vb_hbm, out_hbm, a_buf, b_buf, o_buf, in_sem0, in_sem1, out_sem0, out_sem1):
    cid = lax.axis_index("c")
    sid = lax.axis_index("s")
    wid = sid * 2 + cid
    slot0 = SC_CHUNK0 + wid * SLOTS
    last = NCHUNKS - 1
    in_sems = (in_sem0, in_sem1)
    out_sems = (out_sem0, out_sem1)

    def in_copies(slot, buf):
        c = jnp.minimum(slot, last)
        ca = pltpu.make_async_copy(
            va_hbm.at[:, pl.ds(c * CC, CC)], a_buf.at[buf], in_sems[buf]
        )
        cb = pltpu.make_async_copy(
            vb_hbm.at[:, pl.ds(c * CC, CC)], b_buf.at[buf], in_sems[buf]
        )
        return ca, cb

    def out_copy(slot, buf):
        c = jnp.minimum(slot, last)
        return pltpu.make_async_copy(
            o_buf.at[buf], out_hbm.at[pl.ds(c * OUT, OUT)], out_sems[buf]
        )

    for buf in range(2):
        ca, cb = in_copies(slot0 + buf, buf)
        ca.start()
        cb.start()

    def pair_body(i, _):
        for buf in range(2):
            slot = slot0 + i * 2 + buf
            ca, cb = in_copies(slot, buf)
            ca.wait()
            cb.wait()

            @pl.when(i > 0)
            def _wait_prev_out():
                out_copy(slot - 2, buf).wait()

            def k_body(k, _):
                for t in range(CC):
                    acc = jnp.zeros((16,), jnp.float32)
                    for tr in range(TR):
                        for s in range(8):
                            va = a_buf[buf, tr, t, s, pl.ds(k * 16, 16)]
                            vb = b_buf[buf, tr, t, s, pl.ds(k * 16, 16)]
                            acc = acc + va * vb
                    o_buf[buf, pl.ds(t * 128 + k * 16, 16)] = acc
                return _

            lax.fori_loop(0, 8, k_body, 0)
            out_copy(slot, buf).start()

            @pl.when(i < (SLOTS // 2) - 1)
            def _start_next_in():
                na, nb = in_copies(slot + 2, buf)
                na.start()
                nb.start()

        return _

    lax.fori_loop(0, SLOTS // 2, pair_body, 0)
    for buf in range(2):
        out_copy(slot0 + SLOTS - 2 + buf, buf).wait()


def _tc_body(gu_ref, fi_ref, out_ref):
    p = gu_ref[...] * fi_ref[...]
    out_ref[...] = jnp.sum(p, axis=0)


def kernel(gu, fi):
    guT = gu.T
    fiT = fi.T
    tc_out = pl.pallas_call(
        _tc_body,
        grid=(_TC_GRID,),
        in_specs=[
            pl.BlockSpec((D, _TC_BLK), lambda i: (0, i)),
            pl.BlockSpec((D, _TC_BLK), lambda i: (0, i)),
        ],
        out_specs=pl.BlockSpec((_TC_BLK,), lambda i: (i,)),
        out_shape=jax.ShapeDtypeStruct((B,), jnp.float32),
    )(guT, fiT)
    va = gu.reshape(TCOLS, 128, 8, 8).transpose(2, 0, 3, 1)
    vb = fi.reshape(TCOLS, 128, 8, 8).transpose(2, 0, 3, 1)
    sc_out = _sc_part(va, vb)
    return jnp.concatenate([tc_out[:SPLIT], sc_out[SPLIT:]], axis=0)


# speedup vs baseline: 1.0134x; 1.0134x over previous
import functools

import jax
import jax.numpy as jnp
from jax import lax
from jax.experimental import pallas as pl
from jax.experimental.pallas import tpu as pltpu
from jax.experimental.pallas import tpu_sc as plsc

B = 800000
D = 64
TR = 8
TCOLS = 6250
CC = 2
OUT = CC * 128
NW = 32

_TC_BLK = 16384
_TC_GRID = 40
SPLIT = _TC_BLK * _TC_GRID
SC_CHUNK0 = SPLIT // (CC * 128)
NCHUNKS = B // (CC * 128)
SC_NCH = NCHUNKS - SC_CHUNK0
SLOTS = -(-SC_NCH // NW)
if SLOTS % 2:
    SLOTS += 1

_mesh = plsc.VectorSubcoreMesh(core_axis_name="c", subcore_axis_name="s")


@functools.partial(
    pl.kernel,
    mesh=_mesh,
    out_type=jax.ShapeDtypeStruct((B,), jnp.float32),
    scratch_types=[
        pltpu.VMEM((2, TR, CC, 8, 128), jnp.float32),
        pltpu.VMEM((2, TR, CC, 8, 128), jnp.float32),
        pltpu.VMEM((2, OUT), jnp.float32),
        pltpu.SemaphoreType.DMA,
        pltpu.SemaphoreType.DMA,
        pltpu.SemaphoreType.DMA,
        pltpu.SemaphoreType.DMA,
    ],
)
def _sc_part(va_hbm, vb_hbm, out_hbm, a_buf, b_buf, o_buf, in_sem0, in_sem1, out_sem0, out_sem1):
    cid = lax.axis_index("c")
    sid = lax.axis_index("s")
    wid = sid * 2 + cid
    slot0 = SC_CHUNK0 + wid * SLOTS
    last = NCHUNKS - 1
    in_sems = (in_sem0, in_sem1)
    out_sems = (out_sem0, out_sem1)

    def in_copies(slot, buf):
        c = jnp.minimum(slot, last)
        ca = pltpu.make_async_copy(
            va_hbm.at[:, pl.ds(c * CC, CC)], a_buf.at[buf], in_sems[buf]
        )
        cb = pltpu.make_async_copy(
            vb_hbm.at[:, pl.ds(c * CC, CC)], b_buf.at[buf], in_sems[buf]
        )
        return ca, cb

    def out_copy(slot, buf):
        c = jnp.minimum(slot, last)
        return pltpu.make_async_copy(
            o_buf.at[buf], out_hbm.at[pl.ds(c * OUT, OUT)], out_sems[buf]
        )

    for buf in range(2):
        ca, cb = in_copies(slot0 + buf, buf)
        ca.start()
        cb.start()

    def pair_body(i, _):
        for buf in range(2):
            slot = slot0 + i * 2 + buf
            ca, cb = in_copies(slot, buf)
            ca.wait()
            cb.wait()

            @pl.when(i > 0)
            def _wait_prev_out():
                out_copy(slot - 2, buf).wait()

            def k_body(k, _):
                for t in range(CC):
                    acc = jnp.zeros((16,), jnp.float32)
                    for tr in range(TR):
                        for s in range(8):
                            va = a_buf[buf, tr, t, s, pl.ds(k * 16, 16)]
                            vb = b_buf[buf, tr, t, s, pl.ds(k * 16, 16)]
                            acc = acc + va * vb
                    o_buf[buf, pl.ds(t * 128 + k * 16, 16)] = acc
                return _

            lax.fori_loop(0, 8, k_body, 0)
            out_copy(slot, buf).start()

            @pl.when(i < (SLOTS // 2) - 1)
            def _start_next_in():
                na, nb = in_copies(slot + 2, buf)
                na.start()
                nb.start()

        return _

    lax.fori_loop(0, SLOTS // 2, pair_body, 0)
    for buf in range(2):
        out_copy(slot0 + SLOTS - 2 + buf, buf).wait()


def _tc_body(gu_ref, fi_ref, out_ref):
    p = gu_ref[...] * fi_ref[...]
    out_ref[...] = jnp.sum(p, axis=0)


def kernel(gu, fi):
    guT = gu.T
    fiT = fi.T
    tc_out = pl.pallas_call(
        _tc_body,
        grid=(_TC_GRID,),
        in_specs=[
            pl.BlockSpec((D, _TC_BLK), lambda i: (0, i)),
            pl.BlockSpec((D, _TC_BLK), lambda i: (0, i)),
        ],
        out_specs=pl.BlockSpec((_TC_BLK,), lambda i: (i,)),
        out_shape=jax.ShapeDtypeStruct((B,), jnp.float32),
    )(guT, fiT)
    va = gu.reshape(TCOLS, 128, 8, 8).transpose(2, 0, 3, 1)
    vb = fi.reshape(TCOLS, 128, 8, 8).transpose(2, 0, 3, 1)
    sc_out = _sc_part(va, vb)
    return jnp.concatenate([tc_out[:SPLIT], sc_out[SPLIT:]], axis=0)


# speedup vs baseline: 1.0199x; 1.0064x over previous
import functools

import jax
import jax.numpy as jnp
from jax import lax
from jax.experimental import pallas as pl
from jax.experimental.pallas import tpu as pltpu
from jax.experimental.pallas import tpu_sc as plsc

B = 800000
D = 64
TR = 8
TCOLS = 6250
CC = 2
OUT = CC * 128
NW = 32

_TC_BLK = 16384
_TC_GRID = 48
SPLIT = _TC_BLK * _TC_GRID
SC_CHUNK0 = SPLIT // (CC * 128)
NCHUNKS = B // (CC * 128)
SC_NCH = NCHUNKS - SC_CHUNK0
SLOTS = -(-SC_NCH // NW)
if SLOTS % 2:
    SLOTS += 1

_mesh = plsc.VectorSubcoreMesh(core_axis_name="c", subcore_axis_name="s")


@functools.partial(
    pl.kernel,
    mesh=_mesh,
    out_type=jax.ShapeDtypeStruct((B,), jnp.float32),
    scratch_types=[
        pltpu.VMEM((2, TR, CC, 8, 128), jnp.float32),
        pltpu.VMEM((2, TR, CC, 8, 128), jnp.float32),
        pltpu.VMEM((2, OUT), jnp.float32),
        pltpu.SemaphoreType.DMA,
        pltpu.SemaphoreType.DMA,
        pltpu.SemaphoreType.DMA,
        pltpu.SemaphoreType.DMA,
    ],
)
def _sc_part(va_hbm, vb_hbm, out_hbm, a_buf, b_buf, o_buf, in_sem0, in_sem1, out_sem0, out_sem1):
    cid = lax.axis_index("c")
    sid = lax.axis_index("s")
    wid = sid * 2 + cid
    slot0 = SC_CHUNK0 + wid * SLOTS
    last = NCHUNKS - 1
    in_sems = (in_sem0, in_sem1)
    out_sems = (out_sem0, out_sem1)

    def in_copies(slot, buf):
        c = jnp.minimum(slot, last)
        ca = pltpu.make_async_copy(
            va_hbm.at[:, pl.ds(c * CC, CC)], a_buf.at[buf], in_sems[buf]
        )
        cb = pltpu.make_async_copy(
            vb_hbm.at[:, pl.ds(c * CC, CC)], b_buf.at[buf], in_sems[buf]
        )
        return ca, cb

    def out_copy(slot, buf):
        c = jnp.minimum(slot, last)
        return pltpu.make_async_copy(
            o_buf.at[buf], out_hbm.at[pl.ds(c * OUT, OUT)], out_sems[buf]
        )

    for buf in range(2):
        ca, cb = in_copies(slot0 + buf, buf)
        ca.start()
        cb.start()

    def pair_body(i, _):
        for buf in range(2):
            slot = slot0 + i * 2 + buf
            ca, cb = in_copies(slot, buf)
            ca.wait()
            cb.wait()

            @pl.when(i > 0)
            def _wait_prev_out():
                out_copy(slot - 2, buf).wait()

            def k_body(k, _):
                for t in range(CC):
                    acc = jnp.zeros((16,), jnp.float32)
                    for tr in range(TR):
                        for s in range(8):
                            va = a_buf[buf, tr, t, s, pl.ds(k * 16, 16)]
                            vb = b_buf[buf, tr, t, s, pl.ds(k * 16, 16)]
                            acc = acc + va * vb
                    o_buf[buf, pl.ds(t * 128 + k * 16, 16)] = acc
                return _

            lax.fori_loop(0, 8, k_body, 0)
            out_copy(slot, buf).start()

            @pl.when(i < (SLOTS // 2) - 1)
            def _start_next_in():
                na, nb = in_copies(slot + 2, buf)
                na.start()
                nb.start()

        return _

    lax.fori_loop(0, SLOTS // 2, pair_body, 0)
    for buf in range(2):
        out_copy(slot0 + SLOTS - 2 + buf, buf).wait()


def _tc_body(gu_ref, fi_ref, out_ref):
    p = gu_ref[...] * fi_ref[...]
    out_ref[...] = jnp.sum(p, axis=0)


def kernel(gu, fi):
    guT = gu.T
    fiT = fi.T
    tc_out = pl.pallas_call(
        _tc_body,
        grid=(_TC_GRID,),
        in_specs=[
            pl.BlockSpec((D, _TC_BLK), lambda i: (0, i)),
            pl.BlockSpec((D, _TC_BLK), lambda i: (0, i)),
        ],
        out_specs=pl.BlockSpec((_TC_BLK,), lambda i: (i,)),
        out_shape=jax.ShapeDtypeStruct((B,), jnp.float32),
    )(guT, fiT)
    va = gu.reshape(TCOLS, 128, 8, 8).transpose(2, 0, 3, 1)
    vb = fi.reshape(TCOLS, 128, 8, 8).transpose(2, 0, 3, 1)
    sc_out = _sc_part(va, vb)
    return jnp.concatenate([tc_out[:SPLIT], sc_out[SPLIT:]], axis=0)


# speedup vs baseline: 1.1939x; 1.1706x over previous
import jax
import jax.numpy as jnp
from jax.experimental import pallas as pl

_BLKN = 32768


def _body(gu_ref, fi_ref, out_ref):
    p = gu_ref[...] * fi_ref[...]
    out_ref[...] = jnp.sum(p, axis=0)


def kernel(gu, fi):
    B, D = gu.shape
    grid = pl.cdiv(B, _BLKN)
    out = pl.pallas_call(
        _body,
        grid=(grid,),
        in_specs=[
            pl.BlockSpec((D, _BLKN), lambda i: (0, i)),
            pl.BlockSpec((D, _BLKN), lambda i: (0, i)),
        ],
        out_specs=pl.BlockSpec((_BLKN,), lambda i: (i,)),
        out_shape=jax.ShapeDtypeStruct((B,), jnp.float32),
    )(gu.T, fi.T)
    return out
